# Initial kernel scaffold; baseline (speedup 1.0000x reference)
import jax
import jax.numpy as jnp
from jax.experimental import pallas as pl


def kernel(x, embedding_table, possitional_emb):
    raise NotImplementedError("write your pallas kernel here")


# speedup vs baseline: 1.3928x; 1.3928x over previous
import functools

import jax
import jax.numpy as jnp
from jax import lax
from jax.experimental import pallas as pl
from jax.experimental.pallas import tpu as pltpu
from jax.experimental.pallas import tpu_sc as plsc

D = 32
SEQ = 200
LANES = 16

_info = plsc.get_sparse_core_info()
_NC, _NS = _info.num_cores, _info.num_subcores
_NW = _NC * _NS


@functools.partial(jax.jit, static_argnums=0)
def _embed(n_rows, x_flat, table, pos):
    per_w = n_rows // _NW
    seq_per_chunk = 4
    ch = seq_per_chunk * SEQ
    ng = per_w // ch
    gs = 80

    mesh = plsc.VectorSubcoreMesh(core_axis_name="c", subcore_axis_name="s")

    @functools.partial(
        pl.kernel,
        mesh=mesh,
        out_type=jax.ShapeDtypeStruct((n_rows, D), jnp.float32),
        compiler_params=pltpu.CompilerParams(use_tc_tiling_on_sc=False),
        scratch_types=[
            pltpu.VMEM((ch,), jnp.int32),
            pltpu.VMEM((ch, D), jnp.float32),
            pltpu.VMEM((SEQ, D), jnp.float32),
            pltpu.SemaphoreType.DMA,
        ],
    )
    def k(x_hbm, tab_hbm, pos_hbm, out_hbm, idx_v, rows_v, pos_v, sem):
        wid = lax.axis_index("s") * _NC + lax.axis_index("c")
        base = wid * per_w
        pltpu.sync_copy(pos_hbm, pos_v)

        def chunk_body(g, carry):
            row0 = pl.multiple_of(base + g * ch, 8)
            pltpu.sync_copy(x_hbm.at[pl.ds(row0, ch)], idx_v)
            copies = [
                pltpu.async_copy(
                    tab_hbm.at[idx_v.at[pl.ds(j * gs, gs)]],
                    rows_v.at[pl.ds(j * gs, gs)],
                    sem,
                )
                for j in range(ch // gs)
            ]
            for c in copies:
                c.wait()

            def pos_add(p, c2):
                pv0 = pos_v[p, pl.ds(0, LANES)]
                pv1 = pos_v[p, pl.ds(LANES, LANES)]
                for s in range(seq_per_chunk):
                    r = s * SEQ + p
                    rows_v[r, pl.ds(0, LANES)] = rows_v[r, pl.ds(0, LANES)] + pv0
                    rows_v[r, pl.ds(LANES, LANES)] = (
                        rows_v[r, pl.ds(LANES, LANES)] + pv1
                    )
                return c2

            lax.fori_loop(0, SEQ, pos_add, 0)
            pltpu.sync_copy(rows_v, out_hbm.at[pl.ds(row0, ch)])
            return carry

        lax.fori_loop(0, ng, chunk_body, 0)

    return k(x_flat, table, pos)


def kernel(x, embedding_table, possitional_emb):
    b, l = x.shape
    out = _embed(b * l, x.reshape(b * l), embedding_table, possitional_emb)
    return out.reshape(b, l, D)


# speedup vs baseline: 1.4899x; 1.0697x over previous
import functools

import jax
import jax.numpy as jnp
from jax import lax
from jax.experimental import pallas as pl
from jax.experimental.pallas import tpu as pltpu
from jax.experimental.pallas import tpu_sc as plsc

D = 32
SEQ = 200
LANES = 16

_info = plsc.get_sparse_core_info()
_NC, _NS = _info.num_cores, _info.num_subcores
_NW = _NC * _NS

_NB = 4
_SPC = 4
_CH = _SPC * SEQ
_GS = 80


@functools.partial(jax.jit, static_argnums=0)
def _embed(n_rows, x_flat, table, pos):
    per_w = n_rows // _NW
    ng = per_w // _CH

    mesh = plsc.VectorSubcoreMesh(core_axis_name="c", subcore_axis_name="s")

    scratch = (
        [pltpu.VMEM((_CH,), jnp.int32) for _ in range(_NB)]
        + [pltpu.VMEM((_CH, D), jnp.float32) for _ in range(_NB)]
        + [pltpu.VMEM((SEQ, D), jnp.float32)]
        + [pltpu.SemaphoreType.DMA for _ in range(3 * _NB)]
    )

    @functools.partial(
        pl.kernel,
        mesh=mesh,
        out_type=jax.ShapeDtypeStruct((n_rows, D), jnp.float32),
        compiler_params=pltpu.CompilerParams(use_tc_tiling_on_sc=False),
        scratch_types=scratch,
    )
    def k(x_hbm, tab_hbm, pos_hbm, out_hbm, *sc):
        idx = sc[:_NB]
        rows = sc[_NB:2 * _NB]
        pos_v = sc[2 * _NB]
        isem = sc[2 * _NB + 1:2 * _NB + 1 + _NB]
        gsem = sc[2 * _NB + 1 + _NB:2 * _NB + 1 + 2 * _NB]
        osem = sc[2 * _NB + 1 + 2 * _NB:]

        wid = lax.axis_index("s") * _NC + lax.axis_index("c")
        base = wid * per_w
        pltpu.sync_copy(pos_hbm, pos_v)

        idx_d, gat_d, out_d = {}, {}, {}

        def fire_idx(g):
            b = g % _NB
            row0 = pl.multiple_of(base + g * _CH, 8)
            idx_d[g] = pltpu.async_copy(
                x_hbm.at[pl.ds(row0, _CH)], idx[b], isem[b])

        def fire_gathers(g):
            b = g % _NB
            if g >= _NB:
                out_d.pop(g - _NB).wait()
            idx_d.pop(g).wait()
            gat_d[g] = [
                pltpu.async_copy(
                    tab_hbm.at[idx[b].at[pl.ds(j * _GS, _GS)]],
                    rows[b].at[pl.ds(j * _GS, _GS)],
                    gsem[b],
                )
                for j in range(_CH // _GS)
            ]

        def pos_add(b):
            rb = rows[b]

            def body(p, c):
                pv0 = pos_v[p, pl.ds(0, LANES)]
                pv1 = pos_v[p, pl.ds(LANES, LANES)]
                for s in range(_SPC):
                    r = s * SEQ + p
                    rb[r, pl.ds(0, LANES)] = rb[r, pl.ds(0, LANES)] + pv0
                    rb[r, pl.ds(LANES, LANES)] = (
                        rb[r, pl.ds(LANES, LANES)] + pv1)
                return c

            lax.fori_loop(0, SEQ, body, 0)

        def fire_out(g):
            b = g % _NB
            row0 = pl.multiple_of(base + g * _CH, 8)
            out_d[g] = pltpu.async_copy(
                rows[b], out_hbm.at[pl.ds(row0, _CH)], osem[b])

        for g in range(min(3, ng)):
            fire_idx(g)
        for g in range(min(2, ng)):
            fire_gathers(g)

        for g in range(ng):
            if g + 3 < ng:
                fire_idx(g + 3)
            if g + 2 < ng:
                fire_gathers(g + 2)
            for d in gat_d.pop(g):
                d.wait()
            pos_add(g % _NB)
            fire_out(g)

        for g in sorted(out_d):
            out_d[g].wait()

    return k(x_flat, table, pos)


def kernel(x, embedding_table, possitional_emb):
    b, l = x.shape
    out = _embed(b * l, x.reshape(b * l), embedding_table, possitional_emb)
    return out.reshape(b, l, D)
